# baseline (speedup 1.0000x reference)
import functools

import jax
import jax.numpy as jnp
from jax.experimental import pallas as pl
from jax.experimental.pallas import tpu as pltpu


def _stats_body(x_ref, out_ref):
    xb = x_ref[...]
    s = jnp.sum(xb, axis=0)
    q = jnp.sum(xb * xb, axis=0)
    partial = jnp.stack([s, q])

    @pl.when(pl.program_id(0) == 0)
    def _init():
        out_ref[...] = partial

    @pl.when(pl.program_id(0) != 0)
    def _acc():
        out_ref[...] = out_ref[...] + partial


def _rnn_body(x_ref, stats_ref, bn_ref, Wx_ref, Wh_ref, b_ref, Wd_ref,
              bd_ref, out_ref, h_ref, scale_ref, b2_ref, *, inv_m):
    bidx = pl.program_id(0)
    t = pl.program_id(1)

    @pl.when(jnp.logical_and(bidx == 0, t == 0))
    def _fold_bn():
        mean = stats_ref[0:1, :] * inv_m
        var = stats_ref[1:2, :] * inv_m - mean * mean
        scale = bn_ref[0:1, :] * jax.lax.rsqrt(var + 1e-5)
        shift = bn_ref[1:2, :] - mean * scale
        scale_ref[...] = scale
        b2_ref[...] = b_ref[...] + jnp.dot(
            shift, Wx_ref[...], preferred_element_type=jnp.float32)

    @pl.when(t == 0)
    def _reset_h():
        h_ref[...] = jnp.zeros_like(h_ref)

    xb = x_ref[0, 0] * scale_ref[...]
    pre = (jnp.dot(xb, Wx_ref[...], preferred_element_type=jnp.float32)
           + jnp.dot(h_ref[...], Wh_ref[...],
                     preferred_element_type=jnp.float32)
           + b2_ref[...])
    h = jnp.tanh(pre)
    h_ref[...] = h
    out_ref[0, 0] = jnp.dot(
        h, Wd_ref[...], preferred_element_type=jnp.float32) + bd_ref[...]


def kernel(x, bn_gamma, bn_beta, Wx, Wh, b, Wd, bd):
    B, T, N, F = x.shape
    H = Wh.shape[0]
    O = Wd.shape[1]
    M = B * T * N

    rows = N
    for cand in (16000, 8000, 4000, 2000):
        if M % cand == 0:
            rows = cand
            break
    x2 = x.reshape(M, F)
    stats = pl.pallas_call(
        _stats_body,
        grid=(M // rows,),
        in_specs=[pl.BlockSpec((rows, F), lambda i: (i, 0))],
        out_specs=pl.BlockSpec((2, F), lambda i: (0, 0)),
        out_shape=jax.ShapeDtypeStruct((2, F), jnp.float32),
    )(x2)

    bn = jnp.stack([bn_gamma, bn_beta])
    full = lambda shape: pl.BlockSpec(shape, lambda bi, ti: (0, 0))
    out = pl.pallas_call(
        functools.partial(_rnn_body, inv_m=1.0 / M),
        grid=(B, T),
        in_specs=[
            pl.BlockSpec((1, 1, N, F), lambda bi, ti: (bi, ti, 0, 0)),
            full((2, F)),
            full((2, F)),
            full((F, H)),
            full((H, H)),
            full((1, H)),
            full((H, O)),
            full((1, O)),
        ],
        out_specs=pl.BlockSpec((1, 1, N, O), lambda bi, ti: (bi, ti, 0, 0)),
        out_shape=jax.ShapeDtypeStruct((B, T, N, O), jnp.float32),
        scratch_shapes=[
            pltpu.VMEM((N, H), jnp.float32),
            pltpu.VMEM((1, F), jnp.float32),
            pltpu.VMEM((1, H), jnp.float32),
        ],
    )(x, stats, bn, Wx, Wh, b.reshape(1, H), Wd, bd.reshape(1, O))
    return out


# speedup vs baseline: 1.2347x; 1.2347x over previous
import functools

import jax
import jax.numpy as jnp
from jax.experimental import pallas as pl
from jax.experimental.pallas import tpu as pltpu


def _stats_body(x_ref, out_ref):
    xb = x_ref[...]
    s = jnp.sum(xb, axis=0)
    q = jnp.sum(xb * xb, axis=0)
    partial = jnp.stack([s, q])

    @pl.when(pl.program_id(0) == 0)
    def _init():
        out_ref[...] = partial

    @pl.when(pl.program_id(0) != 0)
    def _acc():
        out_ref[...] = out_ref[...] + partial


def _rnn_body(x_ref, stats_ref, bn_ref, Wx_ref, Wh_ref, b_ref, Wd_ref,
              bd_ref, out_ref, hall_ref, Wx2_ref, b2_ref, *, inv_m, T):
    bidx = pl.program_id(0)
    nidx = pl.program_id(1)

    @pl.when(jnp.logical_and(bidx == 0, nidx == 0))
    def _fold_bn():
        mean = stats_ref[0:1, :] * inv_m
        var = stats_ref[1:2, :] * inv_m - mean * mean
        scale = bn_ref[0:1, :] * jax.lax.rsqrt(var + 1e-5)
        shift = bn_ref[1:2, :] - mean * scale
        Wx2_ref[...] = (Wx_ref[...] * jnp.transpose(scale)).astype(
            jnp.bfloat16)
        b2_ref[...] = b_ref[...] + jnp.dot(
            shift, Wx_ref[...], preferred_element_type=jnp.float32)

    Wx2 = Wx2_ref[...]
    b2 = b2_ref[...]
    h = None
    for t in range(T):
        pre = jnp.dot(x_ref[0, t].astype(jnp.bfloat16), Wx2,
                      preferred_element_type=jnp.float32) + b2
        if h is not None:
            pre = pre + jnp.dot(h.astype(jnp.bfloat16), Wh_ref[...],
                                preferred_element_type=jnp.float32)
        h = jnp.tanh(pre)
        hall_ref[t] = h

    nb = hall_ref.shape[1]
    res = jnp.dot(hall_ref[...].reshape(T * nb, -1).astype(jnp.bfloat16),
                  Wd_ref[...], preferred_element_type=jnp.float32)
    out_ref[0] = (res + bd_ref[...]).reshape(T, nb, -1)


def kernel(x, bn_gamma, bn_beta, Wx, Wh, b, Wd, bd):
    B, T, N, F = x.shape
    H = Wh.shape[0]
    O = Wd.shape[1]
    M = B * T * N

    rows = N
    for cand in (16000, 8000, 4000, 2000):
        if M % cand == 0:
            rows = cand
            break
    x2 = x.reshape(M, F)
    stats = pl.pallas_call(
        _stats_body,
        grid=(M // rows,),
        in_specs=[pl.BlockSpec((rows, F), lambda i: (i, 0))],
        out_specs=pl.BlockSpec((2, F), lambda i: (0, 0)),
        out_shape=jax.ShapeDtypeStruct((2, F), jnp.float32),
    )(x2)

    nb = N
    for cand in (1000, 500, 400, 250):
        if N % cand == 0:
            nb = cand
            break
    bn = jnp.stack([bn_gamma, bn_beta])
    full = lambda shape: pl.BlockSpec(shape, lambda bi, ni: (0, 0))
    out = pl.pallas_call(
        functools.partial(_rnn_body, inv_m=1.0 / M, T=T),
        grid=(B, N // nb),
        in_specs=[
            pl.BlockSpec((1, T, nb, F), lambda bi, ni: (bi, 0, ni, 0)),
            full((2, F)),
            full((2, F)),
            full((F, H)),
            full((H, H)),
            full((1, H)),
            full((H, O)),
            full((1, O)),
        ],
        out_specs=pl.BlockSpec((1, T, nb, O), lambda bi, ni: (bi, 0, ni, 0)),
        out_shape=jax.ShapeDtypeStruct((B, T, N, O), jnp.float32),
        scratch_shapes=[
            pltpu.VMEM((T, nb, H), jnp.float32),
            pltpu.VMEM((F, H), jnp.bfloat16),
            pltpu.VMEM((1, H), jnp.float32),
        ],
        compiler_params=pltpu.CompilerParams(
            vmem_limit_bytes=100 * 1024 * 1024),
    )(x, stats, bn, Wx, Wh.astype(jnp.bfloat16), b.reshape(1, H),
      Wd.astype(jnp.bfloat16), bd.reshape(1, O))
    return out


# speedup vs baseline: 1.3287x; 1.0761x over previous
import functools

import jax
import jax.numpy as jnp
from jax.experimental import pallas as pl
from jax.experimental.pallas import tpu as pltpu


def _stats_body(x_ref, out_ref):
    xb = x_ref[...]
    s = jnp.sum(xb, axis=0)
    q = jnp.sum(xb * xb, axis=0)
    partial = jnp.stack([s, q])

    @pl.when(pl.program_id(0) == 0)
    def _init():
        out_ref[...] = partial

    @pl.when(pl.program_id(0) != 0)
    def _acc():
        out_ref[...] = out_ref[...] + partial


def _rnn_body(x_ref, stats_ref, bn_ref, Wx_ref, Wh_ref, b_ref, Wd_ref,
              bd_ref, out_ref, hall_ref, Wx2_ref, b2_ref, *, inv_m, T):
    bidx = pl.program_id(0)

    @pl.when(bidx == 0)
    def _fold_bn():
        mean = stats_ref[0:1, :] * inv_m
        var = stats_ref[1:2, :] * inv_m - mean * mean
        scale = bn_ref[0:1, :] * jax.lax.rsqrt(var + 1e-5)
        shift = bn_ref[1:2, :] - mean * scale
        Wx2_ref[...] = (Wx_ref[...] * jnp.transpose(scale)).astype(
            jnp.bfloat16)
        b2_ref[...] = b_ref[...] + jnp.dot(
            shift, Wx_ref[...], preferred_element_type=jnp.float32)

    nb = hall_ref.shape[1]
    h_dim = hall_ref.shape[2]
    b2 = b2_ref[...]
    p = jnp.dot(x_ref[0].reshape(T * nb, -1).astype(jnp.bfloat16),
                Wx2_ref[...], preferred_element_type=jnp.float32) + b2
    h = jnp.tanh(p[0:nb])
    hall_ref[0] = h.astype(jnp.bfloat16)
    for t in range(1, T):
        pre = p[t * nb:(t + 1) * nb] + jnp.dot(
            h.astype(jnp.bfloat16), Wh_ref[...],
            preferred_element_type=jnp.float32)
        h = jnp.tanh(pre)
        hall_ref[t] = h.astype(jnp.bfloat16)

    res = jnp.dot(hall_ref[...].reshape(T * nb, h_dim),
                  Wd_ref[...], preferred_element_type=jnp.float32)
    out_ref[0] = (res + bd_ref[...]).reshape(T, nb, -1)


def kernel(x, bn_gamma, bn_beta, Wx, Wh, b, Wd, bd):
    B, T, N, F = x.shape
    H = Wh.shape[0]
    O = Wd.shape[1]
    M = B * T * N

    rows = N
    for cand in (16000, 8000, 4000, 2000):
        if M % cand == 0:
            rows = cand
            break
    x2 = x.reshape(M, F)
    stats = pl.pallas_call(
        _stats_body,
        grid=(M // rows,),
        in_specs=[pl.BlockSpec((rows, F), lambda i: (i, 0))],
        out_specs=pl.BlockSpec((2, F), lambda i: (0, 0)),
        out_shape=jax.ShapeDtypeStruct((2, F), jnp.float32),
    )(x2)

    nb = N
    bn = jnp.stack([bn_gamma, bn_beta])
    full = lambda shape: pl.BlockSpec(shape, lambda bi: (0, 0))
    out = pl.pallas_call(
        functools.partial(_rnn_body, inv_m=1.0 / M, T=T),
        grid=(B,),
        in_specs=[
            pl.BlockSpec((1, T, nb, F), lambda bi: (bi, 0, 0, 0)),
            full((2, F)),
            full((2, F)),
            full((F, H)),
            full((H, H)),
            full((1, H)),
            full((H, O)),
            full((1, O)),
        ],
        out_specs=pl.BlockSpec((1, T, nb, O), lambda bi: (bi, 0, 0, 0)),
        out_shape=jax.ShapeDtypeStruct((B, T, N, O), jnp.float32),
        scratch_shapes=[
            pltpu.VMEM((T, nb, H), jnp.bfloat16),
            pltpu.VMEM((F, H), jnp.bfloat16),
            pltpu.VMEM((1, H), jnp.float32),
        ],
        compiler_params=pltpu.CompilerParams(
            vmem_limit_bytes=100 * 1024 * 1024),
    )(x, stats, bn, Wx, Wh.astype(jnp.bfloat16), b.reshape(1, H),
      Wd.astype(jnp.bfloat16), bd.reshape(1, O))
    return out
